# baseline (speedup 1.0000x reference)
import functools

import jax
import jax.numpy as jnp
from jax import lax
from jax.experimental import pallas as pl
from jax.experimental.pallas import tpu as pltpu
from jax.experimental.pallas import tpu_sc as plsc

_NC = 2
_NS = 16
_NW = _NC * _NS
_SIZES = (64, 64, 128, 256)
_NP = len(_SIZES)


def kernel(idx, table):
    B = idx.shape[0]
    per_w = B // _NW
    assert sum(_SIZES) == per_w

    idx_r = idx.reshape(_NW, per_w)

    mesh = plsc.VectorSubcoreMesh(
        core_axis_name="c", subcore_axis_name="s",
        num_cores=_NC, num_subcores=_NS,
    )

    @functools.partial(
        pl.kernel,
        out_type=jax.ShapeDtypeStruct((_NW, per_w), jnp.int32),
        mesh=mesh,
        scratch_types=[
            pltpu.VMEM((per_w,), jnp.int32),
            pltpu.VMEM((per_w,), jnp.int32),
            pltpu.SemaphoreType.DMA((_NP,)),
            pltpu.SemaphoreType.DMA((_NP,)),
            pltpu.SemaphoreType.DMA,
        ],
    )
    def gather_kernel(table_hbm, idx_hbm, out_hbm, idx_v, vals_v,
                      sem_i, sem_g, sem_o):
        wid = lax.axis_index("s") * _NC + lax.axis_index("c")
        offs = [sum(_SIZES[:j]) for j in range(_NP)]
        sl = [pl.ds(offs[j], _SIZES[j]) for j in range(_NP)]
        h0 = pl.ds(0, offs[2])
        h1 = pl.ds(offs[2], per_w - offs[2])
        ci0 = pltpu.async_copy(idx_hbm.at[wid, h0], idx_v.at[h0], sem_i.at[0])
        ci1 = pltpu.async_copy(idx_hbm.at[wid, h1], idx_v.at[h1], sem_i.at[1])
        gs = []
        ci0.wait()
        for j in range(2):
            gs.append(
                pltpu.async_copy(table_hbm.at[idx_v.at[sl[j]]],
                                 vals_v.at[sl[j]], sem_g.at[j])
            )
        ci1.wait()
        for j in range(2, _NP):
            gs.append(
                pltpu.async_copy(table_hbm.at[idx_v.at[sl[j]]],
                                 vals_v.at[sl[j]], sem_g.at[j])
            )
        gs[0].wait()
        gs[1].wait()
        o0 = pltpu.async_copy(vals_v.at[h0], out_hbm.at[wid, h0], sem_o)
        gs[2].wait()
        gs[3].wait()
        o1 = pltpu.async_copy(vals_v.at[h1], out_hbm.at[wid, h1], sem_o)
        o0.wait()
        o1.wait()
    out = gather_kernel(table, idx_r)
    return out.reshape(B)


# speedup vs baseline: 1.0019x; 1.0019x over previous
import dataclasses
import functools

import jax
import jax.numpy as jnp
from jax import lax
from jax.experimental import pallas as pl
from jax.experimental.pallas import tpu as pltpu
from jax.experimental.pallas import tpu_sc as plsc
from jax._src.pallas import mpmd
from jax._src.pallas import core as pallas_core
from jax._src.pallas.mosaic import core as tpu_core

_NC = 2
_NS = 16
_NW = _NC * _NS
_SIZES = (64, 64, 128, 256)
_NP = len(_SIZES)


def _vq(mem_ref, mesh):
    return dataclasses.replace(
        mem_ref,
        memory_space=pallas_core.CoreMemorySpace(mem_ref.memory_space, mesh),
    )


def kernel(idx, table):
    B = idx.shape[0]
    per_w = B // _NW
    assert sum(_SIZES) == per_w

    idx_r = idx.reshape(_NW, per_w)

    scalar_mesh = plsc.ScalarSubcoreMesh(axis_name="c", num_cores=_NC)
    vector_mesh = plsc.VectorSubcoreMesh(
        core_axis_name="c", subcore_axis_name="s",
        num_cores=_NC, num_subcores=_NS,
    )

    scratch_types = [
        _vq(pltpu.VMEM((per_w,), jnp.int32), vector_mesh),
        _vq(pltpu.VMEM((per_w,), jnp.int32), vector_mesh),
        _vq(pltpu.SemaphoreType.DMA((_NP,)), vector_mesh),
        _vq(pltpu.SemaphoreType.DMA((_NP,)), vector_mesh),
        _vq(pltpu.SemaphoreType.DMA(()), vector_mesh),
    ]

    def scs_fn(table_hbm, idx_hbm, out_hbm, idx_v, vals_v, sem_i, sem_g, sem_o):
        del table_hbm, idx_hbm, out_hbm, idx_v, vals_v, sem_i, sem_g, sem_o

    def tec_fn(table_hbm, idx_hbm, out_hbm, idx_v, vals_v, sem_i, sem_g, sem_o):
        wid = lax.axis_index("s") * _NC + lax.axis_index("c")
        offs = [sum(_SIZES[:j]) for j in range(_NP)]
        sl = [pl.ds(offs[j], _SIZES[j]) for j in range(_NP)]
        ci = [
            pltpu.async_copy(idx_hbm.at[wid, sl[j]], idx_v.at[sl[j]],
                             sem_i.at[j])
            for j in range(_NP)
        ]
        gs = []
        for j in range(_NP):
            ci[j].wait()
            gs.append(
                pltpu.async_copy(table_hbm.at[idx_v.at[sl[j]]],
                                 vals_v.at[sl[j]], sem_g.at[j])
            )
        os = []
        for j in range(_NP):
            gs[j].wait()
            os.append(
                pltpu.async_copy(vals_v.at[sl[j]], out_hbm.at[wid, sl[j]],
                                 sem_o)
            )
        for o in os:
            o.wait()

    run = mpmd.mpmd_map(
        [(scalar_mesh, scs_fn), (vector_mesh, tec_fn)],
        out_types=jax.ShapeDtypeStruct((_NW, per_w), jnp.int32),
        scratch_types=scratch_types,
    )
    out = run(table, idx_r)
    return out.reshape(B)


# speedup vs baseline: 1.0126x; 1.0107x over previous
import dataclasses
import functools

import jax
import jax.numpy as jnp
from jax import lax
from jax.experimental import pallas as pl
from jax.experimental.pallas import tpu as pltpu
from jax.experimental.pallas import tpu_sc as plsc
from jax._src.pallas import mpmd
from jax._src.pallas import core as pallas_core
from jax._src.pallas.mosaic import core as tpu_core

_NC = 2
_NS = 16
_NW = _NC * _NS
_SIZES = (64, 64, 128, 256)
_NP = len(_SIZES)


def _vq(mem_ref, mesh):
    return dataclasses.replace(
        mem_ref,
        memory_space=pallas_core.CoreMemorySpace(mem_ref.memory_space, mesh),
    )


def kernel(idx, table):
    B = idx.shape[0]
    per_w = B // _NW
    assert sum(_SIZES) == per_w

    idx_r = idx.reshape(_NC, _NS, per_w)

    scalar_mesh = plsc.ScalarSubcoreMesh(axis_name="c", num_cores=_NC)
    vector_mesh = plsc.VectorSubcoreMesh(
        core_axis_name="c", subcore_axis_name="s",
        num_cores=_NC, num_subcores=_NS,
    )

    scratch_types = [
        pltpu.VMEM_SHARED((_NS, per_w), jnp.int32),
        _vq(pltpu.SemaphoreType.REGULAR(()), vector_mesh),
        _vq(pltpu.VMEM((per_w,), jnp.int32), vector_mesh),
        _vq(pltpu.VMEM((per_w,), jnp.int32), vector_mesh),
        _vq(pltpu.SemaphoreType.DMA((_NP,)), vector_mesh),
        _vq(pltpu.SemaphoreType.DMA(()), vector_mesh),
    ]

    def scs_fn(table_hbm, idx_hbm, out_hbm, idx_sh, ready, idx_v, vals_v,
               sem_g, sem_o):
        del table_hbm, out_hbm, idx_v, vals_v, sem_g, sem_o
        cid = lax.axis_index("c")
        pltpu.sync_copy(idx_hbm.at[cid], idx_sh)
        for s in range(_NS):
            pltpu.semaphore_signal(ready, 1, device_id={"s": s})

    def tec_fn(table_hbm, idx_hbm, out_hbm, idx_sh, ready, idx_v, vals_v,
               sem_g, sem_o):
        del idx_hbm
        cid = lax.axis_index("c")
        sid = lax.axis_index("s")
        offs = [sum(_SIZES[:j]) for j in range(_NP)]
        sl = [pl.ds(offs[j], _SIZES[j]) for j in range(_NP)]
        pl.semaphore_wait(ready, 1)
        pltpu.sync_copy(idx_sh.at[sid], idx_v)
        gs = []
        for j in range(_NP):
            gs.append(
                pltpu.async_copy(table_hbm.at[idx_v.at[sl[j]]],
                                 vals_v.at[sl[j]], sem_g.at[j])
            )
        os = []
        for j in range(_NP):
            gs[j].wait()
            os.append(
                pltpu.async_copy(vals_v.at[sl[j]],
                                 out_hbm.at[cid, sid, sl[j]], sem_o)
            )
        for o in os:
            o.wait()

    run = mpmd.mpmd_map(
        [(scalar_mesh, scs_fn), (vector_mesh, tec_fn)],
        out_types=jax.ShapeDtypeStruct((_NC, _NS, per_w), jnp.int32),
        scratch_types=scratch_types,
    )
    out = run(table, idx_r)
    return out.reshape(B)
